# baseline (speedup 1.0000x reference)
import functools

import jax
import jax.numpy as jnp
from jax import lax
from jax.experimental import pallas as pl
from jax.experimental.pallas import tpu as pltpu, tpu_sc as plsc

N = 100000
G = 5000
NUM_SUBCORES = 16
SLAB = 6272
N_PAD = SLAB * NUM_SUBCORES
HALF = SLAB // 2
G_ROWS = 40
CHUNKS_ACC = SLAB // 16
CHUNKS_OUT = HALF // 16


def _body(e_hbm, s_hbm, ids_hbm, zeros_hbm, iota_hbm, out_hbm,
          ids_v, e_v, s_v, sinv_v, acc_s, acc_e, sum_s, sum_e, iota_v, outq_v,
          sh_s, sh_e):
    sid = lax.axis_index("s")
    base = sid * SLAB

    with jax.named_scope("stage_in"):
        pltpu.sync_copy(e_hbm.at[pl.ds(base, SLAB)], e_v)
        pltpu.sync_copy(s_hbm.at[pl.ds(base, SLAB)], s_v)
        pltpu.sync_copy(ids_hbm.at[pl.ds(base, SLAB)], ids_v)
        pltpu.sync_copy(zeros_hbm, acc_s)
        pltpu.sync_copy(zeros_hbm, acc_e)
        pltpu.sync_copy(iota_hbm, iota_v)

        @pl.when(sid == 0)
        def _():
            pltpu.sync_copy(zeros_hbm, sh_s)
            pltpu.sync_copy(zeros_hbm, sh_e)

        plsc.subcore_barrier()

    def acc_body(i, carry):
        off = i * 16
        ids16 = ids_v[pl.ds(off, 16)]
        e16 = e_v[pl.ds(off, 16)]
        s16 = s_v[pl.ds(off, 16)]
        sinv = 1.0 / s16
        es = e16 * sinv
        sinv_v[pl.ds(off, 16)] = sinv
        row = lax.shift_right_logical(ids16, 7)
        col = jnp.bitwise_and(ids16, 127)
        plsc.addupdate_scatter(acc_s, [row, col], sinv)
        plsc.addupdate_scatter(acc_e, [row, col], es)
        return carry

    with jax.named_scope("acc_loop"):
        lax.fori_loop(0, CHUNKS_ACC, acc_body, 0)

    with jax.named_scope("merge"):
        pltpu.sync_copy(acc_s, sh_s.at[iota_v], add=True)
        pltpu.sync_copy(acc_e, sh_e.at[iota_v], add=True)

        plsc.subcore_barrier()

    with jax.named_scope("sums_back"):
        pltpu.sync_copy(sh_s, sum_s)
        pltpu.sync_copy(sh_e, sum_e)

    def out_body(i, carry):
        off = i * 16
        ids16 = ids_v[pl.ds(off, 16)]
        row = lax.shift_right_logical(ids16, 7)
        col = jnp.bitwise_and(ids16, 127)
        g_s = plsc.load_gather(sum_s, [row, col])
        g_e = plsc.load_gather(sum_e, [row, col])
        sinv = sinv_v[pl.ds(off, 16)]
        e16 = e_v[pl.ds(off, 16)]
        q = sinv * (g_e / g_s) - e16 * sinv
        outq_v[pl.ds(i * 16, 16)] = q
        return carry

    with jax.named_scope("out_loop"):
        lax.fori_loop(0, CHUNKS_ACC, out_body, 0)

    with jax.named_scope("out_write"):
        pltpu.sync_copy(outq_v, out_hbm.at[pl.ds(base, SLAB)])


@functools.partial(
    pl.kernel,
    out_type=jax.ShapeDtypeStruct((N_PAD,), jnp.float32),
    mesh=plsc.VectorSubcoreMesh(core_axis_name="c", subcore_axis_name="s", num_cores=1),
    compiler_params=pltpu.CompilerParams(needs_layout_passes=False),
    scratch_types=[
        pltpu.VMEM((SLAB,), jnp.int32),
        pltpu.VMEM((SLAB,), jnp.float32),
        pltpu.VMEM((SLAB,), jnp.float32),
        pltpu.VMEM((SLAB,), jnp.float32),
        pltpu.VMEM((G_ROWS, 128), jnp.float32),
        pltpu.VMEM((G_ROWS, 128), jnp.float32),
        pltpu.VMEM((G_ROWS, 128), jnp.float32),
        pltpu.VMEM((G_ROWS, 128), jnp.float32),
        pltpu.VMEM((G_ROWS,), jnp.int32),
        pltpu.VMEM((SLAB,), jnp.float32),
        pltpu.VMEM_SHARED((G_ROWS, 128), jnp.float32),
        pltpu.VMEM_SHARED((G_ROWS, 128), jnp.float32),
    ],
)
def _sc_kernel(e_hbm, s_hbm, ids_hbm, zeros_hbm, iota_hbm, out_hbm, *scratch):
    _body(e_hbm, s_hbm, ids_hbm, zeros_hbm, iota_hbm, out_hbm, *scratch)


def kernel(e, s, segment_ids):
    pad = N_PAD - N
    e1 = jnp.concatenate([e.reshape(-1), jnp.zeros((pad,), jnp.float32)])
    s1 = jnp.concatenate([s.reshape(-1), jnp.ones((pad,), jnp.float32)])
    ids1 = jnp.concatenate(
        [segment_ids, jnp.full((pad,), G, jnp.int32)])
    zeros = jnp.zeros((G_ROWS, 128), jnp.float32)
    iota = jnp.arange(G_ROWS, dtype=jnp.int32)
    q = _sc_kernel(e1, s1, ids1, zeros, iota)
    return q[:N].reshape(N, 1)


# speedup vs baseline: 1.2061x; 1.2061x over previous
import functools

import jax
import jax.numpy as jnp
from jax import lax
from jax.experimental import pallas as pl
from jax.experimental.pallas import tpu as pltpu, tpu_sc as plsc

N = 100000
G = 5000
NUM_SUBCORES = 16
SLAB = 6272
LAST = N - 15 * SLAB
G_ROWS = 48
ROWS_PER_TILE = G_ROWS // NUM_SUBCORES
UNROLL = 4
SUPER_FULL = SLAB // (16 * UNROLL)
SUPER_LAST = LAST // (16 * UNROLL)
TAIL_CHUNKS = LAST // 16 - SUPER_LAST * UNROLL


def _body(e_hbm, s_hbm, ids_hbm, out_hbm,
          ids_v, e_v, s_v, sinv_v, acc_s, acc_e, ratio3, rs3, re3,
          ratio_v, iota_v, outq_v, sh_s, sh_e, sh_r, sem):
    sid = lax.axis_index("s")
    base = sid * SLAB
    last = sid == NUM_SUBCORES - 1

    with jax.named_scope("stage_in"):
        n_in = jnp.where(last, LAST, SLAB)
        cp_e = pltpu.async_copy(e_hbm.at[pl.ds(base, n_in)],
                                e_v.at[pl.ds(0, n_in)], sem)
        cp_s = pltpu.async_copy(s_hbm.at[pl.ds(base, n_in)],
                                s_v.at[pl.ds(0, n_in)], sem)
        cp_i = pltpu.async_copy(ids_hbm.at[pl.ds(base, n_in)],
                                ids_v.at[pl.ds(0, n_in)], sem)

    with jax.named_scope("zero_scratch"):
        zf = jnp.zeros((16,), jnp.float32)

        def zrow(r, carry):
            for k in range(8):
                acc_s[r, pl.ds(k * 16, 16)] = zf
                acc_e[r, pl.ds(k * 16, 16)] = zf
            return carry

        lax.fori_loop(0, G_ROWS, zrow, 0)
        it16 = lax.iota(jnp.int32, 16)
        for j in range(G_ROWS // 16):
            iota_v[pl.ds(j * 16, 16)] = it16 + (16 * j)

        @pl.when(sid == 0)
        def _():
            pltpu.sync_copy(acc_s, sh_s)
            pltpu.sync_copy(acc_e, sh_e)

    plsc.subcore_barrier()

    with jax.named_scope("stage_wait"):
        cp_e.wait()
        cp_s.wait()
        cp_i.wait()

        @pl.when(last)
        def _():
            idg = jnp.full((16,), G, jnp.int32)
            on = jnp.ones((16,), jnp.float32)
            for j in range(SLAB // 16 - LAST // 16):
                off = LAST + j * 16
                e_v[pl.ds(off, 16)] = zf
                s_v[pl.ds(off, 16)] = on
                ids_v[pl.ds(off, 16)] = idg

    def acc_chunk(off):
        ids16 = ids_v[pl.ds(off, 16)]
        e16 = e_v[pl.ds(off, 16)]
        s16 = s_v[pl.ds(off, 16)]
        sinv = 1.0 / s16
        es = e16 * sinv
        sinv_v[pl.ds(off, 16)] = sinv
        row = lax.shift_right_logical(ids16, 7)
        col = jnp.bitwise_and(ids16, 127)
        plsc.addupdate_scatter(acc_s, [row, col], sinv)
        plsc.addupdate_scatter(acc_e, [row, col], es)

    n_super = jnp.where(last, SUPER_LAST, SUPER_FULL)

    with jax.named_scope("acc_loop"):
        def acc_body(i, carry):
            for k in range(UNROLL):
                acc_chunk(i * (16 * UNROLL) + k * 16)
            return carry

        lax.fori_loop(0, n_super, acc_body, 0)

        @pl.when(last)
        def _():
            for k in range(TAIL_CHUNKS):
                acc_chunk(SUPER_LAST * 16 * UNROLL + k * 16)

    with jax.named_scope("merge"):
        pltpu.sync_copy(acc_s, sh_s.at[iota_v], add=True)
        pltpu.sync_copy(acc_e, sh_e.at[iota_v], add=True)

    plsc.subcore_barrier()

    with jax.named_scope("ratio"):
        r0 = sid * ROWS_PER_TILE
        pltpu.sync_copy(sh_s.at[pl.ds(r0, ROWS_PER_TILE)], rs3)
        pltpu.sync_copy(sh_e.at[pl.ds(r0, ROWS_PER_TILE)], re3)
        for r in range(ROWS_PER_TILE):
            for k in range(8):
                ratio3[r, pl.ds(k * 16, 16)] = (
                    re3[r, pl.ds(k * 16, 16)] / rs3[r, pl.ds(k * 16, 16)])
        pltpu.sync_copy(ratio3, sh_r.at[pl.ds(r0, ROWS_PER_TILE)])

    plsc.subcore_barrier()

    with jax.named_scope("ratio_back"):
        pltpu.sync_copy(sh_r, ratio_v)

    def out_chunk(off):
        ids16 = ids_v[pl.ds(off, 16)]
        row = lax.shift_right_logical(ids16, 7)
        col = jnp.bitwise_and(ids16, 127)
        g_r = plsc.load_gather(ratio_v, [row, col])
        sinv = sinv_v[pl.ds(off, 16)]
        e16 = e_v[pl.ds(off, 16)]
        outq_v[pl.ds(off, 16)] = sinv * (g_r - e16)

    with jax.named_scope("out_loop"):
        def out_body(i, carry):
            for k in range(UNROLL):
                out_chunk(i * (16 * UNROLL) + k * 16)
            return carry

        lax.fori_loop(0, n_super, out_body, 0)

        @pl.when(last)
        def _():
            for k in range(TAIL_CHUNKS):
                out_chunk(SUPER_LAST * 16 * UNROLL + k * 16)

    with jax.named_scope("out_write"):
        n_out = jnp.where(last, LAST, SLAB)
        pltpu.sync_copy(outq_v.at[pl.ds(0, n_out)],
                        out_hbm.at[pl.ds(base, n_out)])


@functools.partial(
    pl.kernel,
    out_type=jax.ShapeDtypeStruct((N,), jnp.float32),
    mesh=plsc.VectorSubcoreMesh(core_axis_name="c", subcore_axis_name="s",
                                num_cores=1),
    compiler_params=pltpu.CompilerParams(needs_layout_passes=False),
    scratch_types=[
        pltpu.VMEM((SLAB,), jnp.int32),
        pltpu.VMEM((SLAB,), jnp.float32),
        pltpu.VMEM((SLAB,), jnp.float32),
        pltpu.VMEM((SLAB,), jnp.float32),
        pltpu.VMEM((G_ROWS, 128), jnp.float32),
        pltpu.VMEM((G_ROWS, 128), jnp.float32),
        pltpu.VMEM((ROWS_PER_TILE, 128), jnp.float32),
        pltpu.VMEM((ROWS_PER_TILE, 128), jnp.float32),
        pltpu.VMEM((ROWS_PER_TILE, 128), jnp.float32),
        pltpu.VMEM((G_ROWS, 128), jnp.float32),
        pltpu.VMEM((G_ROWS,), jnp.int32),
        pltpu.VMEM((SLAB,), jnp.float32),
        pltpu.VMEM_SHARED((G_ROWS, 128), jnp.float32),
        pltpu.VMEM_SHARED((G_ROWS, 128), jnp.float32),
        pltpu.VMEM_SHARED((G_ROWS, 128), jnp.float32),
        pltpu.SemaphoreType.DMA,
    ],
)
def _sc_kernel(e_hbm, s_hbm, ids_hbm, out_hbm, *scratch):
    _body(e_hbm, s_hbm, ids_hbm, out_hbm, *scratch)


def kernel(e, s, segment_ids):
    q = _sc_kernel(e.reshape(-1), s.reshape(-1), segment_ids)
    return q.reshape(N, 1)


# speedup vs baseline: 1.2068x; 1.0006x over previous
import functools

import jax
import jax.numpy as jnp
from jax import lax
from jax.experimental import pallas as pl
from jax.experimental.pallas import tpu as pltpu, tpu_sc as plsc

N = 100000
G = 5000
NUM_SUBCORES = 16
SLAB = 6272
LAST = N - 15 * SLAB
G_ROWS = 48
ROWS_PER_TILE = G_ROWS // NUM_SUBCORES
UNROLL = 4
SUPER_FULL = SLAB // (16 * UNROLL)
SUPER_LAST = LAST // (16 * UNROLL)
TAIL_CHUNKS = LAST // 16 - SUPER_LAST * UNROLL


def _body(e_hbm, s_hbm, ids_hbm, out_hbm,
          ids_v, e_v, s_v, sinv_v, acc_s, acc_e, ratio3, rs3, re3,
          ratio_v, iota_v, outq_v, sh_s, sh_e, sh_r, sem):
    sid = lax.axis_index("s")
    base = sid * SLAB
    last = sid == NUM_SUBCORES - 1

    with jax.named_scope("stage_in"):
        n_in = jnp.where(last, LAST, SLAB)
        cp_e = pltpu.async_copy(e_hbm.at[pl.ds(base, n_in)],
                                e_v.at[pl.ds(0, n_in)], sem)
        cp_s = pltpu.async_copy(s_hbm.at[pl.ds(base, n_in)],
                                s_v.at[pl.ds(0, n_in)], sem)
        cp_i = pltpu.async_copy(ids_hbm.at[pl.ds(base, n_in)],
                                ids_v.at[pl.ds(0, n_in)], sem)

    with jax.named_scope("zero_scratch"):
        zf = jnp.zeros((16,), jnp.float32)

        def zrow(r, carry):
            for k in range(8):
                acc_s[r, pl.ds(k * 16, 16)] = zf
                acc_e[r, pl.ds(k * 16, 16)] = zf
            return carry

        lax.fori_loop(0, G_ROWS, zrow, 0)
        it16 = lax.iota(jnp.int32, 16)
        for j in range(G_ROWS // 16):
            iota_v[pl.ds(j * 16, 16)] = it16 + (16 * j)

        @pl.when(sid == 0)
        def _():
            pltpu.sync_copy(acc_s, sh_s)
            pltpu.sync_copy(acc_e, sh_e)

    plsc.subcore_barrier()

    with jax.named_scope("stage_wait"):
        cp_e.wait()
        cp_s.wait()
        cp_i.wait()

        @pl.when(last)
        def _():
            on = jnp.ones((16,), jnp.float32)
            idg = jnp.full((16,), G, jnp.int32)
            for j in range(SLAB // 16 - LAST // 16):
                off = LAST + j * 16
                e_v[pl.ds(off, 16)] = zf
                s_v[pl.ds(off, 16)] = on
                ids_v[pl.ds(off, 16)] = idg

    def acc_chunk(off):
        ids16 = ids_v[pl.ds(off, 16)]
        e16 = e_v[pl.ds(off, 16)]
        s16 = s_v[pl.ds(off, 16)]
        sinv = 1.0 / s16
        es = e16 * sinv
        sinv_v[pl.ds(off, 16)] = sinv
        row = lax.shift_right_logical(ids16, 7)
        col = jnp.bitwise_and(ids16, 127)
        plsc.addupdate_scatter(acc_s, [row, col], sinv)
        plsc.addupdate_scatter(acc_e, [row, col], es)

    n_super = jnp.where(last, SUPER_LAST, SUPER_FULL)

    with jax.named_scope("acc_loop"):
        def acc_body(i, carry):
            for k in range(UNROLL):
                acc_chunk(i * (16 * UNROLL) + k * 16)
            return carry

        lax.fori_loop(0, n_super, acc_body, 0)

        @pl.when(last)
        def _():
            for k in range(TAIL_CHUNKS):
                acc_chunk(SUPER_LAST * 16 * UNROLL + k * 16)

    with jax.named_scope("merge"):
        pltpu.sync_copy(acc_s, sh_s.at[iota_v], add=True)
        pltpu.sync_copy(acc_e, sh_e.at[iota_v], add=True)

    plsc.subcore_barrier()

    with jax.named_scope("ratio"):
        r0 = sid * ROWS_PER_TILE
        pltpu.sync_copy(sh_s.at[pl.ds(r0, ROWS_PER_TILE)], rs3)
        pltpu.sync_copy(sh_e.at[pl.ds(r0, ROWS_PER_TILE)], re3)
        for r in range(ROWS_PER_TILE):
            for k in range(8):
                ratio3[r, pl.ds(k * 16, 16)] = (
                    re3[r, pl.ds(k * 16, 16)] / rs3[r, pl.ds(k * 16, 16)])
        pltpu.sync_copy(ratio3, sh_r.at[pl.ds(r0, ROWS_PER_TILE)])

    plsc.subcore_barrier()

    with jax.named_scope("ratio_back"):
        pltpu.sync_copy(sh_r, ratio_v)

    def out_chunk(off):
        ids16 = ids_v[pl.ds(off, 16)]
        row = lax.shift_right_logical(ids16, 7)
        col = jnp.bitwise_and(ids16, 127)
        g_r = plsc.load_gather(ratio_v, [row, col])
        sinv = sinv_v[pl.ds(off, 16)]
        e16 = e_v[pl.ds(off, 16)]
        outq_v[pl.ds(off, 16)] = sinv * (g_r - e16)

    with jax.named_scope("out_loop"):
        def out_body(i, carry):
            for k in range(UNROLL):
                out_chunk(i * (16 * UNROLL) + k * 16)
            return carry

        lax.fori_loop(0, n_super, out_body, 0)

        @pl.when(last)
        def _():
            for k in range(TAIL_CHUNKS):
                out_chunk(SUPER_LAST * 16 * UNROLL + k * 16)

    with jax.named_scope("out_write"):
        n_out = jnp.where(last, LAST, SLAB)
        pltpu.sync_copy(outq_v.at[pl.ds(0, n_out)],
                        out_hbm.at[pl.ds(base, n_out)])


@functools.partial(
    pl.kernel,
    out_type=jax.ShapeDtypeStruct((N,), jnp.float32),
    mesh=plsc.VectorSubcoreMesh(core_axis_name="c", subcore_axis_name="s",
                                num_cores=1),
    compiler_params=pltpu.CompilerParams(needs_layout_passes=False),
    scratch_types=[
        pltpu.VMEM((SLAB,), jnp.int32),
        pltpu.VMEM((SLAB,), jnp.float32),
        pltpu.VMEM((SLAB,), jnp.float32),
        pltpu.VMEM((SLAB,), jnp.float32),
        pltpu.VMEM((G_ROWS, 128), jnp.float32),
        pltpu.VMEM((G_ROWS, 128), jnp.float32),
        pltpu.VMEM((ROWS_PER_TILE, 128), jnp.float32),
        pltpu.VMEM((ROWS_PER_TILE, 128), jnp.float32),
        pltpu.VMEM((ROWS_PER_TILE, 128), jnp.float32),
        pltpu.VMEM((G_ROWS, 128), jnp.float32),
        pltpu.VMEM((G_ROWS,), jnp.int32),
        pltpu.VMEM((SLAB,), jnp.float32),
        pltpu.VMEM_SHARED((G_ROWS, 128), jnp.float32),
        pltpu.VMEM_SHARED((G_ROWS, 128), jnp.float32),
        pltpu.VMEM_SHARED((G_ROWS, 128), jnp.float32),
        pltpu.SemaphoreType.DMA,
    ],
)
def _sc_kernel(e_hbm, s_hbm, ids_hbm, out_hbm, *scratch):
    _body(e_hbm, s_hbm, ids_hbm, out_hbm, *scratch)


def kernel(e, s, segment_ids):
    q = _sc_kernel(e.reshape(-1), s.reshape(-1), segment_ids)
    return q.reshape(N, 1)


# speedup vs baseline: 1.4331x; 1.1875x over previous
import functools

import jax
import jax.numpy as jnp
from jax import lax
from jax.experimental import pallas as pl
from jax.experimental.pallas import tpu as pltpu, tpu_sc as plsc

N = 100000
G = 5000
NUM_SUBCORES = 16
SLAB = 6272
LAST = N - 15 * SLAB
G_ROWS = 48
ROWS_PER_TILE = G_ROWS // NUM_SUBCORES
UNROLL = 4
SUPER_FULL = SLAB // (16 * UNROLL)
SUPER_LAST = LAST // (16 * UNROLL)
TAIL_CHUNKS = LAST // 16 - SUPER_LAST * UNROLL


def _body(e_hbm, s_hbm, ids_hbm, out_hbm,
          ids_v, e_v, s_v, sinv_v, acc_s, acc_e, ratio3, rs3, re3,
          ratio_v, iota_v, outq_v, sh_s, sh_e, sh_r, sem):
    sid = lax.axis_index("s")
    base = sid * SLAB
    last = sid == NUM_SUBCORES - 1

    with jax.named_scope("stage_in"):
        n_in = jnp.where(last, LAST, SLAB)
        cp_e = pltpu.async_copy(e_hbm.at[pl.ds(base, n_in)],
                                e_v.at[pl.ds(0, n_in)], sem)
        cp_s = pltpu.async_copy(s_hbm.at[pl.ds(base, n_in)],
                                s_v.at[pl.ds(0, n_in)], sem)
        cp_i = pltpu.async_copy(ids_hbm.at[pl.ds(base, n_in)],
                                ids_v.at[pl.ds(0, n_in)], sem)

    with jax.named_scope("zero_scratch"):
        zf = jnp.zeros((16,), jnp.float32)

        def zrow(r, carry):
            for k in range(8):
                acc_s[r, pl.ds(k * 16, 16)] = zf
                acc_e[r, pl.ds(k * 16, 16)] = zf
            return carry

        lax.fori_loop(0, G_ROWS, zrow, 0)
        it16 = lax.iota(jnp.int32, 16)
        for j in range(G_ROWS // 16):
            iota_v[pl.ds(j * 16, 16)] = it16 + (16 * j)

        @pl.when(sid == 0)
        def _():
            pltpu.sync_copy(acc_s, sh_s)
            pltpu.sync_copy(acc_e, sh_e)

    plsc.subcore_barrier()

    with jax.named_scope("stage_wait"):
        cp_e.wait()
        cp_s.wait()
        cp_i.wait()
        idg = jnp.full((16,), G, jnp.int32)
        ids_v[pl.ds(SLAB, 16)] = idg

        @pl.when(last)
        def _():
            on = jnp.ones((16,), jnp.float32)
            for j in range(SLAB // 16 - LAST // 16):
                off = LAST + j * 16
                e_v[pl.ds(off, 16)] = zf
                s_v[pl.ds(off, 16)] = on
                ids_v[pl.ds(off, 16)] = idg

    def acc_chunk(off):
        ids16 = ids_v[pl.ds(off, 16)]
        e16 = e_v[pl.ds(off, 16)]
        s16 = s_v[pl.ds(off, 16)]
        sinv = 1.0 / s16
        es = e16 * sinv
        sinv_v[pl.ds(off, 16)] = sinv
        nxtm = ids_v[pl.ds(off + 1, 16)]
        diff = ids16 != nxtm
        emask = jnp.logical_or(diff, it16 == 15)
        mask2 = jnp.logical_and(diff, it16 != 15)
        c_s = plsc.cumsum(sinv)
        c_e = plsc.cumsum(es)
        row = lax.shift_right_logical(ids16, 7)
        col = jnp.bitwise_and(ids16, 127)
        plsc.addupdate_scatter(acc_s, [row, col], c_s, mask=emask)
        plsc.addupdate_scatter(acc_e, [row, col], c_e, mask=emask)
        rown = lax.shift_right_logical(nxtm, 7)
        coln = jnp.bitwise_and(nxtm, 127)
        plsc.addupdate_scatter(acc_s, [rown, coln], -c_s, mask=mask2)
        plsc.addupdate_scatter(acc_e, [rown, coln], -c_e, mask=mask2)

    n_super = jnp.where(last, SUPER_LAST, SUPER_FULL)

    with jax.named_scope("acc_loop"):
        def acc_body(i, carry):
            for k in range(UNROLL):
                acc_chunk(i * (16 * UNROLL) + k * 16)
            return carry

        lax.fori_loop(0, n_super, acc_body, 0)

        @pl.when(last)
        def _():
            for k in range(TAIL_CHUNKS):
                acc_chunk(SUPER_LAST * 16 * UNROLL + k * 16)

    with jax.named_scope("merge"):
        pltpu.sync_copy(acc_s, sh_s.at[iota_v], add=True)
        pltpu.sync_copy(acc_e, sh_e.at[iota_v], add=True)

    plsc.subcore_barrier()

    with jax.named_scope("ratio"):
        r0 = sid * ROWS_PER_TILE
        pltpu.sync_copy(sh_s.at[pl.ds(r0, ROWS_PER_TILE)], rs3)
        pltpu.sync_copy(sh_e.at[pl.ds(r0, ROWS_PER_TILE)], re3)
        for r in range(ROWS_PER_TILE):
            for k in range(8):
                ratio3[r, pl.ds(k * 16, 16)] = (
                    re3[r, pl.ds(k * 16, 16)] / rs3[r, pl.ds(k * 16, 16)])
        pltpu.sync_copy(ratio3, sh_r.at[pl.ds(r0, ROWS_PER_TILE)])

    plsc.subcore_barrier()

    with jax.named_scope("ratio_back"):
        pltpu.sync_copy(sh_r, ratio_v)

    def out_chunk(off):
        ids16 = ids_v[pl.ds(off, 16)]
        row = lax.shift_right_logical(ids16, 7)
        col = jnp.bitwise_and(ids16, 127)
        g_r = plsc.load_gather(ratio_v, [row, col])
        sinv = sinv_v[pl.ds(off, 16)]
        e16 = e_v[pl.ds(off, 16)]
        outq_v[pl.ds(off, 16)] = sinv * (g_r - e16)

    with jax.named_scope("out_loop"):
        def out_body(i, carry):
            for k in range(UNROLL):
                out_chunk(i * (16 * UNROLL) + k * 16)
            return carry

        lax.fori_loop(0, n_super, out_body, 0)

        @pl.when(last)
        def _():
            for k in range(TAIL_CHUNKS):
                out_chunk(SUPER_LAST * 16 * UNROLL + k * 16)

    with jax.named_scope("out_write"):
        n_out = jnp.where(last, LAST, SLAB)
        pltpu.sync_copy(outq_v.at[pl.ds(0, n_out)],
                        out_hbm.at[pl.ds(base, n_out)])


@functools.partial(
    pl.kernel,
    out_type=jax.ShapeDtypeStruct((N,), jnp.float32),
    mesh=plsc.VectorSubcoreMesh(core_axis_name="c", subcore_axis_name="s",
                                num_cores=1),
    compiler_params=pltpu.CompilerParams(needs_layout_passes=False),
    scratch_types=[
        pltpu.VMEM((SLAB + 16,), jnp.int32),
        pltpu.VMEM((SLAB,), jnp.float32),
        pltpu.VMEM((SLAB,), jnp.float32),
        pltpu.VMEM((SLAB,), jnp.float32),
        pltpu.VMEM((G_ROWS, 128), jnp.float32),
        pltpu.VMEM((G_ROWS, 128), jnp.float32),
        pltpu.VMEM((ROWS_PER_TILE, 128), jnp.float32),
        pltpu.VMEM((ROWS_PER_TILE, 128), jnp.float32),
        pltpu.VMEM((ROWS_PER_TILE, 128), jnp.float32),
        pltpu.VMEM((G_ROWS, 128), jnp.float32),
        pltpu.VMEM((G_ROWS,), jnp.int32),
        pltpu.VMEM((SLAB,), jnp.float32),
        pltpu.VMEM_SHARED((G_ROWS, 128), jnp.float32),
        pltpu.VMEM_SHARED((G_ROWS, 128), jnp.float32),
        pltpu.VMEM_SHARED((G_ROWS, 128), jnp.float32),
        pltpu.SemaphoreType.DMA,
    ],
)
def _sc_kernel(e_hbm, s_hbm, ids_hbm, out_hbm, *scratch):
    _body(e_hbm, s_hbm, ids_hbm, out_hbm, *scratch)


def kernel(e, s, segment_ids):
    q = _sc_kernel(e.reshape(-1), s.reshape(-1), segment_ids)
    return q.reshape(N, 1)
